# baseline (speedup 1.0000x reference)
import functools

import jax
import jax.numpy as jnp
from jax import lax
from jax.experimental import pallas as pl
from jax.experimental.pallas import tpu as pltpu
from jax.experimental.pallas import tpu_sc as plsc

_BATCH = 4096
_HIST = 50
_D = 64
_NC = 2
_NS = 16
_NW = _NC * _NS
_RPW = _BATCH // _NW
_NB = 8

_mesh = plsc.VectorSubcoreMesh(core_axis_name="c", subcore_axis_name="s")


@functools.partial(
    pl.kernel,
    mesh=_mesh,
    out_type=jax.ShapeDtypeStruct((_BATCH, _HIST, _D), jnp.float32),
    compiler_params=pltpu.CompilerParams(use_tc_tiling_on_sc=False),
    scratch_types=[
        pltpu.VMEM((_RPW, _HIST), jnp.int32),
        pltpu.VMEM((_NB * _HIST, _D), jnp.float32),
    ] + [pltpu.SemaphoreType.DMA] * _NB,
)
def _gather(idx_hbm, table_hbm, out_hbm, idx_v, rows_v, *sems):
    wid = lax.axis_index("s") * _NC + lax.axis_index("c")
    row0 = wid * _RPW
    pltpu.sync_copy(idx_hbm.at[pl.ds(row0, _RPW)], idx_v)

    def buf(b):
        return rows_v.at[pl.ds(b * _HIST, _HIST)]

    for b in range(_NB):
        pltpu.async_copy(table_hbm.at[idx_v.at[b]], buf(b), sems[b])

    def grp(g, carry):
        for b in range(_NB):
            chunk = g * _NB + b
            pltpu.make_async_copy(table_hbm.at[idx_v.at[b]], buf(b), sems[b]).wait()
            pltpu.sync_copy(buf(b), out_hbm.at[row0 + chunk])
            pltpu.async_copy(table_hbm.at[idx_v.at[chunk + _NB]], buf(b), sems[b])
        return carry

    lax.fori_loop(0, _RPW // _NB - 1, grp, 0)

    for b in range(_NB):
        chunk = _RPW - _NB + b
        pltpu.make_async_copy(table_hbm.at[idx_v.at[b]], buf(b), sems[b]).wait()
        pltpu.sync_copy(buf(b), out_hbm.at[row0 + chunk])


def kernel(word_batch, table):
    return _gather(word_batch.astype(jnp.int32), table)
